# baseline (speedup 1.0000x reference)
import functools
import jax
import jax.numpy as jnp
from jax import lax
from jax.experimental import pallas as pl
from jax.experimental.pallas import tpu as pltpu
from jax.experimental.pallas import tpu_sc as plsc

NUM_USERS = 25000
NUM_ITEMS = 25000
N_NODES = NUM_USERS + NUM_ITEMS
N_PAD = 50048
D = 64
H = 32
E = 800000
B = 16384
N_LAYERS = 3

NTEC = 16
CH = 128
S = 14
NSUPER = 28
NCHUNK = S * NSUPER
EPW = NCHUNK * CH
E_PAD = NTEC * EPW
ROWS_PER_TEC = N_PAD // NTEC

_mesh = plsc.VectorSubcoreMesh(core_axis_name="c", subcore_axis_name="s")
_params = pltpu.CompilerParams(use_tc_tiling_on_sc=False,
                               needs_layout_passes=False)


@functools.partial(
    pl.kernel,
    mesh=_mesh,
    out_type=(
        jax.ShapeDtypeStruct((N_PAD, H), jnp.float32),
        jax.ShapeDtypeStruct((N_PAD, H), jnp.float32),
    ),
    scratch_types=[
        pltpu.VMEM((2 * S, CH), jnp.int32),
        pltpu.VMEM((S, CH), jnp.float32),
        pltpu.VMEM((CH, H), jnp.float32),
        pltpu.VMEM((CH, H), jnp.float32),
        pltpu.VMEM_SHARED((N_PAD, H), jnp.float32),
        pltpu.SemaphoreType.DMA,
        pltpu.SemaphoreType.DMA,
        pltpu.SemaphoreType.DMA,
        pltpu.SemaphoreType.DMA,
    ],
    compiler_params=_params,
)
def _spmm_layer(xL, xR, zeros_h, idx_h, val_h,
                yL, yR, ebuf, vbuf, rows0, rows1, acc_sh,
                gsem0, gsem1, ssem0, ssem1):
    cid = lax.axis_index("c")
    sid = lax.axis_index("s")
    rows = (rows0, rows1)
    gsem = (gsem0, gsem1)
    ssem = (ssem0, ssem1)

    def body(x_hbm, y_hbm):
        r0 = sid * ROWS_PER_TEC
        pltpu.sync_copy(zeros_h, acc_sh.at[pl.ds(r0, ROWS_PER_TEC)])
        plsc.subcore_barrier()

        def super_chunk(u, _):
            base2 = (sid * NSUPER + u) * (2 * S)
            basev = (sid * NSUPER + u) * S
            pltpu.sync_copy(idx_h.at[pl.ds(base2, 2 * S)], ebuf)
            pltpu.sync_copy(val_h.at[pl.ds(basev, S)], vbuf)
            gh = {}
            sh = {}
            gh[0] = pltpu.async_copy(x_hbm.at[ebuf.at[1]], rows[0], gsem[0])
            for j in range(S):
                p = j % 2
                q = 1 - p
                gh[p].wait()
                if j + 1 < S:
                    if j >= 1:
                        sh[q].wait()
                    gh[q] = pltpu.async_copy(
                        x_hbm.at[ebuf.at[2 * (j + 1) + 1]], rows[q], gsem[q])
                @plsc.parallel_loop(0, CH // 16, unroll=2)
                def group(g, p=p, j=j):
                    vvv = vbuf[j, pl.ds(g * 16, 16)]
                    for k in range(16):
                        lane = jnp.full((16,), k, jnp.int32)
                        vv = lax.gather(
                            vvv, lane[:, None],
                            lax.GatherDimensionNumbers(
                                offset_dims=(),
                                collapsed_slice_dims=(0,),
                                start_index_map=(0,)),
                            slice_sizes=(1,),
                            mode=lax.GatherScatterMode.PROMISE_IN_BOUNDS)
                        e = g * 16 + k
                        a = rows[p][e, pl.ds(0, 16)]
                        b = rows[p][e, pl.ds(16, 16)]
                        rows[p][e, pl.ds(0, 16)] = a * vv
                        rows[p][e, pl.ds(16, 16)] = b * vv
                sh[p] = pltpu.async_copy(rows[p], acc_sh.at[ebuf.at[2 * j]],
                                         ssem[p], add=True)
            sh[0].wait()
            sh[1].wait()
            return 0

        lax.fori_loop(0, NSUPER, super_chunk, 0)
        plsc.subcore_barrier()
        pltpu.sync_copy(acc_sh.at[pl.ds(r0, ROWS_PER_TEC)],
                        y_hbm.at[pl.ds(r0, ROWS_PER_TEC)])

    @pl.when(cid == 0)
    def _():
        body(xL, yL)

    @pl.when(cid == 1)
    def _():
        body(xR, yR)


BCH = 128
NW = 2 * NTEC
B_PER_W = B // NW
NBCHUNK = B_PER_W // BCH


@functools.partial(
    pl.kernel,
    mesh=_mesh,
    out_type=jax.ShapeDtypeStruct((B,), jnp.float32),
    scratch_types=[
        pltpu.VMEM((BCH,), jnp.int32),
        pltpu.VMEM((BCH,), jnp.int32),
        pltpu.VMEM((BCH, H), jnp.float32),
        pltpu.VMEM((BCH, H), jnp.float32),
        pltpu.VMEM((BCH, H), jnp.float32),
        pltpu.VMEM((BCH, H), jnp.float32),
        pltpu.VMEM((BCH, H), jnp.float32),
        pltpu.VMEM((BCH,), jnp.float32),
        pltpu.SemaphoreType.DMA,
    ],
    compiler_params=_params,
)
def _gamma_kernel(x0L, x0R, x1L, x1R, x2L, x2R, x3L, x3R, users_h, items_h,
                  out_h, uidx_v, iidx_v, tmp_v, suL_v, suR_v, siL_v, siR_v,
                  g_v, sem):
    cid = lax.axis_index("c")
    sid = lax.axis_index("s")
    wid = sid * 2 + cid
    base = wid * B_PER_W

    def accum(table, idx_v, dest_v, first):
        pltpu.async_copy(table.at[idx_v], tmp_v, sem).wait()

        def row(e, _):
            for half in range(2):
                s = pl.ds(half * 16, 16)
                t = tmp_v[e, s]
                if first:
                    dest_v[e, s] = t
                else:
                    dest_v[e, s] = dest_v[e, s] + t
            return 0

        lax.fori_loop(0, BCH, row, 0)

    def chunk(i, _):
        off = base + i * BCH
        pltpu.sync_copy(users_h.at[pl.ds(off, BCH)], uidx_v)
        pltpu.sync_copy(items_h.at[pl.ds(off, BCH)], iidx_v)
        accum(x0L, uidx_v, suL_v, True)
        accum(x1L, uidx_v, suL_v, False)
        accum(x2L, uidx_v, suL_v, False)
        accum(x3L, uidx_v, suL_v, False)
        accum(x0R, uidx_v, suR_v, True)
        accum(x1R, uidx_v, suR_v, False)
        accum(x2R, uidx_v, suR_v, False)
        accum(x3R, uidx_v, suR_v, False)
        accum(x0L, iidx_v, siL_v, True)
        accum(x1L, iidx_v, siL_v, False)
        accum(x2L, iidx_v, siL_v, False)
        accum(x3L, iidx_v, siL_v, False)
        accum(x0R, iidx_v, siR_v, True)
        accum(x1R, iidx_v, siR_v, False)
        accum(x2R, iidx_v, siR_v, False)
        accum(x3R, iidx_v, siR_v, False)

        def dot(e, _):
            p = (suL_v[e, pl.ds(0, 16)] * siL_v[e, pl.ds(0, 16)]
                 + suL_v[e, pl.ds(16, 16)] * siL_v[e, pl.ds(16, 16)]
                 + suR_v[e, pl.ds(0, 16)] * siR_v[e, pl.ds(0, 16)]
                 + suR_v[e, pl.ds(16, 16)] * siR_v[e, pl.ds(16, 16)])
            s = jnp.sum(p, axis=0) * (1.0 / 16.0)
            plsc.store_scatter(
                g_v,
                [jnp.broadcast_to(e, (16,))],
                jnp.broadcast_to(s, (16,)),
                mask=lax.iota(jnp.int32, 16) == 0,
            )
            return 0

        lax.fori_loop(0, BCH, dot, 0)
        pltpu.sync_copy(g_v, out_h.at[pl.ds(off, BCH)])
        return 0

    lax.fori_loop(0, NBCHUNK, chunk, 0)


def kernel(users, items, user_emb, item_emb, edge_index, edge_vals):
    x0 = jnp.concatenate(
        [user_emb, item_emb, jnp.zeros((N_PAD - N_NODES, D), jnp.float32)],
        axis=0)
    x0L = x0[:, :H]
    x0R = x0[:, H:]

    pad = E_PAD - E
    dst = jnp.concatenate([edge_index[0], jnp.zeros((pad,), jnp.int32)])
    src = jnp.concatenate([edge_index[1], jnp.zeros((pad,), jnp.int32)])
    val = jnp.concatenate([edge_vals, jnp.zeros((pad,), jnp.float32)])
    dst_r = dst.reshape(NTEC, NSUPER, S, 1, CH)
    src_r = src.reshape(NTEC, NSUPER, S, 1, CH)
    idx_h = jnp.concatenate([dst_r, src_r], axis=3).reshape(-1, CH)
    val_h = val.reshape(-1, CH)
    zeros = jnp.zeros((ROWS_PER_TEC, H), jnp.float32)

    xs = [(x0L, x0R)]
    for _ in range(N_LAYERS):
        yL, yR = _spmm_layer(xs[-1][0], xs[-1][1], zeros, idx_h, val_h)
        xs.append((yL, yR))

    items_n = items + NUM_USERS
    gamma = _gamma_kernel(xs[0][0], xs[0][1], xs[1][0], xs[1][1],
                          xs[2][0], xs[2][1], xs[3][0], xs[3][1],
                          users, items_n)
    return gamma


# speedup vs baseline: 1.5186x; 1.5186x over previous
import functools
import jax
import jax.numpy as jnp
from jax import lax
from jax.experimental import pallas as pl
from jax.experimental.pallas import tpu as pltpu
from jax.experimental.pallas import tpu_sc as plsc

NUM_USERS = 25000
NUM_ITEMS = 25000
N_NODES = NUM_USERS + NUM_ITEMS
N_PAD = 50048
D = 64
H = 32
E = 800000
B = 16384
N_LAYERS = 3

NTEC = 16
CH = 128
S = 28
NSUPER = 14
DBUF = 4
GAHEAD = DBUF - 2
NCHUNK = S * NSUPER
EPW = NCHUNK * CH
E_PAD = NTEC * EPW
ROWS_PER_TEC = N_PAD // NTEC

_mesh = plsc.VectorSubcoreMesh(core_axis_name="c", subcore_axis_name="s")
_params = pltpu.CompilerParams(use_tc_tiling_on_sc=False,
                               needs_layout_passes=False)


@functools.partial(
    pl.kernel,
    mesh=_mesh,
    out_type=(
        jax.ShapeDtypeStruct((N_PAD, H), jnp.float32),
        jax.ShapeDtypeStruct((N_PAD, H), jnp.float32),
    ),
    scratch_types=[
        pltpu.VMEM((2 * S, CH), jnp.int32),
        pltpu.VMEM((S, CH), jnp.float32),
        pltpu.VMEM((CH, H), jnp.float32),
        pltpu.VMEM((CH, H), jnp.float32),
        pltpu.VMEM((CH, H), jnp.float32),
        pltpu.VMEM((CH, H), jnp.float32),
        pltpu.VMEM_SHARED((N_PAD, H), jnp.float32),
        pltpu.SemaphoreType.DMA,
        pltpu.SemaphoreType.DMA,
        pltpu.SemaphoreType.DMA,
        pltpu.SemaphoreType.DMA,
        pltpu.SemaphoreType.DMA,
        pltpu.SemaphoreType.DMA,
        pltpu.SemaphoreType.DMA,
        pltpu.SemaphoreType.DMA,
    ],
    compiler_params=_params,
)
def _spmm_layer(xL, xR, zeros_h, idx_h, val_h,
                yL, yR, ebuf, vbuf, rows0, rows1, rows2, rows3,
                acc_sh, gsem0, gsem1, gsem2, gsem3,
                ssem0, ssem1, ssem2, ssem3):
    cid = lax.axis_index("c")
    sid = lax.axis_index("s")
    rows = (rows0, rows1, rows2, rows3)
    gsem = (gsem0, gsem1, gsem2, gsem3)
    ssem = (ssem0, ssem1, ssem2, ssem3)

    def body(x_hbm, y_hbm):
        r0 = sid * ROWS_PER_TEC
        pltpu.sync_copy(zeros_h, acc_sh.at[pl.ds(r0, ROWS_PER_TEC)])
        plsc.subcore_barrier()

        def super_chunk(u, _):
            base2 = (sid * NSUPER + u) * (2 * S)
            basev = (sid * NSUPER + u) * S
            pltpu.sync_copy(idx_h.at[pl.ds(base2, 2 * S)], ebuf)
            pltpu.sync_copy(val_h.at[pl.ds(basev, S)], vbuf)
            gh = {}
            sh = {}
            for c in range(min(GAHEAD, S)):
                b = c % DBUF
                gh[b] = pltpu.async_copy(
                    x_hbm.at[ebuf.at[2 * c + 1]], rows[b], gsem[b])
            for j in range(S):
                b = j % DBUF
                g = j + GAHEAD
                if g < S:
                    gb = g % DBUF
                    if g >= DBUF:
                        sh[gb].wait()
                    gh[gb] = pltpu.async_copy(
                        x_hbm.at[ebuf.at[2 * g + 1]], rows[gb], gsem[gb])
                gh[b].wait()
                j16 = jnp.full((16,), j, jnp.int32)

                @plsc.parallel_loop(0, CH, unroll=8)
                def edge(e, b=b, j16=j16):
                    vv = plsc.load_gather(
                        vbuf, [j16, jnp.broadcast_to(e, (16,))])
                    a = rows[b][e, pl.ds(0, 16)]
                    c2 = rows[b][e, pl.ds(16, 16)]
                    rows[b][e, pl.ds(0, 16)] = a * vv
                    rows[b][e, pl.ds(16, 16)] = c2 * vv
                sh[b] = pltpu.async_copy(rows[b], acc_sh.at[ebuf.at[2 * j]],
                                         ssem[b], add=True)
            for c in range(max(0, S - DBUF), S):
                sh[c % DBUF].wait()
            return 0

        lax.fori_loop(0, NSUPER, super_chunk, 0)
        plsc.subcore_barrier()
        pltpu.sync_copy(acc_sh.at[pl.ds(r0, ROWS_PER_TEC)],
                        y_hbm.at[pl.ds(r0, ROWS_PER_TEC)])

    @pl.when(cid == 0)
    def _():
        body(xL, yL)

    @pl.when(cid == 1)
    def _():
        body(xR, yR)


BCH = 128
NW = 2 * NTEC
B_PER_W = B // NW
NBCHUNK = B_PER_W // BCH


@functools.partial(
    pl.kernel,
    mesh=_mesh,
    out_type=jax.ShapeDtypeStruct((B,), jnp.float32),
    scratch_types=[
        pltpu.VMEM((BCH,), jnp.int32),
        pltpu.VMEM((BCH,), jnp.int32),
        pltpu.VMEM((BCH, H), jnp.float32),
        pltpu.VMEM((BCH, H), jnp.float32),
        pltpu.VMEM((BCH, H), jnp.float32),
        pltpu.VMEM((BCH, H), jnp.float32),
        pltpu.VMEM((BCH, H), jnp.float32),
        pltpu.VMEM((BCH,), jnp.float32),
        pltpu.SemaphoreType.DMA,
    ],
    compiler_params=_params,
)
def _gamma_kernel(x0L, x0R, x1L, x1R, x2L, x2R, x3L, x3R, users_h, items_h,
                  out_h, uidx_v, iidx_v, tmp_v, suL_v, suR_v, siL_v, siR_v,
                  g_v, sem):
    cid = lax.axis_index("c")
    sid = lax.axis_index("s")
    wid = sid * 2 + cid
    base = wid * B_PER_W

    def accum(table, idx_v, dest_v, first):
        pltpu.async_copy(table.at[idx_v], tmp_v, sem).wait()

        def row(e, _):
            for half in range(2):
                s = pl.ds(half * 16, 16)
                t = tmp_v[e, s]
                if first:
                    dest_v[e, s] = t
                else:
                    dest_v[e, s] = dest_v[e, s] + t
            return 0

        lax.fori_loop(0, BCH, row, 0)

    def chunk(i, _):
        off = base + i * BCH
        pltpu.sync_copy(users_h.at[pl.ds(off, BCH)], uidx_v)
        pltpu.sync_copy(items_h.at[pl.ds(off, BCH)], iidx_v)
        accum(x0L, uidx_v, suL_v, True)
        accum(x1L, uidx_v, suL_v, False)
        accum(x2L, uidx_v, suL_v, False)
        accum(x3L, uidx_v, suL_v, False)
        accum(x0R, uidx_v, suR_v, True)
        accum(x1R, uidx_v, suR_v, False)
        accum(x2R, uidx_v, suR_v, False)
        accum(x3R, uidx_v, suR_v, False)
        accum(x0L, iidx_v, siL_v, True)
        accum(x1L, iidx_v, siL_v, False)
        accum(x2L, iidx_v, siL_v, False)
        accum(x3L, iidx_v, siL_v, False)
        accum(x0R, iidx_v, siR_v, True)
        accum(x1R, iidx_v, siR_v, False)
        accum(x2R, iidx_v, siR_v, False)
        accum(x3R, iidx_v, siR_v, False)

        def dot(e, _):
            p = (suL_v[e, pl.ds(0, 16)] * siL_v[e, pl.ds(0, 16)]
                 + suL_v[e, pl.ds(16, 16)] * siL_v[e, pl.ds(16, 16)]
                 + suR_v[e, pl.ds(0, 16)] * siR_v[e, pl.ds(0, 16)]
                 + suR_v[e, pl.ds(16, 16)] * siR_v[e, pl.ds(16, 16)])
            s = jnp.sum(p, axis=0) * (1.0 / 16.0)
            plsc.store_scatter(
                g_v,
                [jnp.broadcast_to(e, (16,))],
                jnp.broadcast_to(s, (16,)),
                mask=lax.iota(jnp.int32, 16) == 0,
            )
            return 0

        lax.fori_loop(0, BCH, dot, 0)
        pltpu.sync_copy(g_v, out_h.at[pl.ds(off, BCH)])
        return 0

    lax.fori_loop(0, NBCHUNK, chunk, 0)


def kernel(users, items, user_emb, item_emb, edge_index, edge_vals):
    x0 = jnp.concatenate(
        [user_emb, item_emb, jnp.zeros((N_PAD - N_NODES, D), jnp.float32)],
        axis=0)
    x0L = x0[:, :H]
    x0R = x0[:, H:]

    pad = E_PAD - E
    dst = jnp.concatenate([edge_index[0], jnp.zeros((pad,), jnp.int32)])
    src = jnp.concatenate([edge_index[1], jnp.zeros((pad,), jnp.int32)])
    val = jnp.concatenate([edge_vals, jnp.zeros((pad,), jnp.float32)])
    dst_r = dst.reshape(NTEC, NSUPER, S, 1, CH)
    src_r = src.reshape(NTEC, NSUPER, S, 1, CH)
    idx_h = jnp.concatenate([dst_r, src_r], axis=3).reshape(-1, CH)
    val_h = val.reshape(-1, CH)
    zeros = jnp.zeros((ROWS_PER_TEC, H), jnp.float32)

    xs = [(x0L, x0R)]
    for _ in range(N_LAYERS):
        yL, yR = _spmm_layer(xs[-1][0], xs[-1][1], zeros, idx_h, val_h)
        xs.append((yL, yR))

    items_n = items + NUM_USERS
    gamma = _gamma_kernel(xs[0][0], xs[0][1], xs[1][0], xs[1][1],
                          xs[2][0], xs[2][1], xs[3][0], xs[3][1],
                          users, items_n)
    return gamma
